# Initial kernel scaffold; baseline (speedup 1.0000x reference)
import jax
import jax.numpy as jnp
from jax.experimental import pallas as pl


def kernel(x, edge_index, W1, b1, W2, b2):
    raise NotImplementedError("write your pallas kernel here")


# speedup vs baseline: 11.2758x; 11.2758x over previous
import functools

import jax
import jax.numpy as jnp
from jax import lax
from jax.experimental import pallas as pl
from jax.experimental.pallas import tpu as pltpu
from jax.experimental.pallas import tpu_sc as plsc

N = 10000
D = 256
HID = 512
C = 40
CP = 48
E = 160000
EPAD = 163840
NC = 2
NS = 16
SUB = 128
NPT = N // NS
TRASH = N
ACCROWS = N + 8

_mesh = plsc.VectorSubcoreMesh(
    core_axis_name="c", subcore_axis_name="s", num_cores=NC, num_subcores=NS
)

_SROWS = 2 * EPAD // SUB
_DROWS = EPAD // SUB


@functools.partial(
    pl.kernel,
    out_type=jax.ShapeDtypeStruct((NC, N, 8), jnp.float32),
    mesh=_mesh,
    compiler_params=pltpu.CompilerParams(use_tc_tiling_on_sc=False),
    scratch_types=[
        pltpu.VMEM((8, SUB), jnp.int32),
        pltpu.VMEM((SUB, 8), jnp.float32),
        pltpu.VMEM_SHARED((ACCROWS, 8), jnp.float32),
    ],
)
def _sc_deg(dst2d, zeros8, ones8, out, didx, ones_v, acc):
    c = lax.axis_index("c")
    s = lax.axis_index("s")
    w = c * NS + s
    pltpu.sync_copy(ones8, ones_v)
    pltpu.sync_copy(zeros8.at[pl.ds(s * NPT, NPT)], acc.at[pl.ds(s * NPT, NPT)])
    plsc.subcore_barrier()

    def body(j, carry):
        pltpu.sync_copy(dst2d.at[pl.ds(w * 40 + j * 8, 8)], didx)
        for k in range(8):
            pltpu.sync_copy(ones_v, acc.at[didx.at[k]], add=True)
        return carry

    lax.fori_loop(0, EPAD // (NC * NS) // (8 * SUB), body, 0)
    plsc.subcore_barrier()
    pltpu.sync_copy(acc.at[pl.ds(s * NPT, NPT)], out.at[c, pl.ds(s * NPT, NPT)])


@functools.partial(
    pl.kernel,
    out_type=jax.ShapeDtypeStruct((NC, N, 128), jnp.float32),
    mesh=_mesh,
    compiler_params=pltpu.CompilerParams(use_tc_tiling_on_sc=False),
    scratch_types=[
        pltpu.VMEM((8, SUB), jnp.int32),
        pltpu.VMEM((8, SUB), jnp.int32),
        pltpu.VMEM((SUB, 128), jnp.float32),
        pltpu.VMEM_SHARED((ACCROWS, 128), jnp.float32),
        pltpu.SemaphoreType.DMA,
    ],
)
def _sc_agg1(t1, src2d, dst2d, out, sidx, didx, rows, acc, sem):
    c = lax.axis_index("c")
    s = lax.axis_index("s")
    pltpu.sync_copy(t1.at[pl.ds(c * N + s * NPT, NPT)], acc.at[pl.ds(s * NPT, NPT)])
    plsc.subcore_barrier()

    def body(j, carry):
        rb = s * 80 + j * 8
        pltpu.sync_copy(src2d.at[pl.ds(c * _DROWS + rb, 8)], sidx)
        pltpu.sync_copy(dst2d.at[pl.ds(rb, 8)], didx)
        for k in range(8):
            pltpu.async_copy(t1.at[sidx.at[k]], rows, sem).wait()
            pltpu.sync_copy(rows, acc.at[didx.at[k]], add=True)
        return carry

    lax.fori_loop(0, 10, body, 0)
    plsc.subcore_barrier()
    pltpu.sync_copy(acc.at[pl.ds(s * NPT, NPT)], out.at[c, pl.ds(s * NPT, NPT)])


@functools.partial(
    pl.kernel,
    out_type=jax.ShapeDtypeStruct((NC, N, CP), jnp.float32),
    mesh=_mesh,
    compiler_params=pltpu.CompilerParams(use_tc_tiling_on_sc=False),
    scratch_types=[
        pltpu.VMEM((8, SUB), jnp.int32),
        pltpu.VMEM((8, SUB), jnp.int32),
        pltpu.VMEM((SUB, CP), jnp.float32),
        pltpu.VMEM_SHARED((ACCROWS, CP), jnp.float32),
        pltpu.SemaphoreType.DMA,
    ],
)
def _sc_agg2(z2p, zeros48, src2d, dst2d, out, sidx, didx, rows, acc, sem):
    c = lax.axis_index("c")
    s = lax.axis_index("s")

    @pl.when(c == 0)
    def _():
        pltpu.sync_copy(z2p.at[pl.ds(s * NPT, NPT)], acc.at[pl.ds(s * NPT, NPT)])

    @pl.when(c != 0)
    def _():
        pltpu.sync_copy(zeros48.at[pl.ds(s * NPT, NPT)], acc.at[pl.ds(s * NPT, NPT)])

    plsc.subcore_barrier()

    def body(j, carry):
        rb = c * 640 + s * 40 + j * 8
        pltpu.sync_copy(src2d.at[pl.ds(rb, 8)], sidx)
        pltpu.sync_copy(dst2d.at[pl.ds(rb, 8)], didx)
        for k in range(8):
            pltpu.async_copy(z2p.at[sidx.at[k]], rows, sem).wait()
            pltpu.sync_copy(rows, acc.at[didx.at[k]], add=True)
        return carry

    lax.fori_loop(0, 5, body, 0)
    plsc.subcore_barrier()
    pltpu.sync_copy(acc.at[pl.ds(s * NPT, NPT)], out.at[c, pl.ds(s * NPT, NPT)])


def _dinv_from(degp_ref):
    deg = 1.0 + degp_ref[0, :, 0:1] + degp_ref[1, :, 0:1]
    return lax.rsqrt(deg)


def _tc_scale_body(x_ref, degp_ref, out_ref):
    out_ref[...] = x_ref[...] * _dinv_from(degp_ref)


def _tc_mid_body(agg_ref, degp_ref, w1_ref, b1_ref, w2_ref, out_ref):
    dinv = _dinv_from(degp_ref)
    ax = jnp.concatenate([agg_ref[0], agg_ref[1]], axis=1) * dinv
    h = jnp.dot(ax, w1_ref[...], preferred_element_type=jnp.float32) + b1_ref[...]
    h = jnp.maximum(h, 0.0)
    out_ref[...] = jnp.dot(h, w2_ref[...], preferred_element_type=jnp.float32) * dinv


def _tc_out_body(agg_ref, degp_ref, b2_ref, out_ref):
    dinv = _dinv_from(degp_ref)
    ssum = (agg_ref[0] + agg_ref[1]) * dinv
    logits = ssum[:, :C] + b2_ref[...]
    m = jnp.max(logits, axis=1, keepdims=True)
    lse = jnp.log(jnp.sum(jnp.exp(logits - m), axis=1, keepdims=True))
    out_ref[...] = logits - m - lse


_RS = 2000


def kernel(x, edge_index, W1, b1, W2, b2):
    src = edge_index[0].astype(jnp.int32)
    dst = edge_index[1].astype(jnp.int32)
    npad = EPAD - E
    src_p = jnp.concatenate([src, jnp.zeros((npad,), jnp.int32)])
    dst_p = jnp.concatenate([dst, jnp.full((npad,), TRASH, jnp.int32)])
    src2d = jnp.concatenate([src_p, src_p + N]).reshape(_SROWS, SUB)
    dst2d = dst_p.reshape(_DROWS, SUB)
    zeros8 = jnp.zeros((N, 8), jnp.float32)
    ones8 = jnp.ones((SUB, 8), jnp.float32)
    zeros48 = jnp.zeros((N, CP), jnp.float32)
    W2p = jnp.concatenate([W2, jnp.zeros((HID, CP - C), jnp.float32)], axis=1)

    degp = _sc_deg(dst2d, zeros8, ones8)

    t1 = pl.pallas_call(
        _tc_scale_body,
        grid=(2, N // _RS),
        in_specs=[
            pl.BlockSpec((_RS, 128), lambda c, i: (i, c)),
            pl.BlockSpec((NC, _RS, 8), lambda c, i: (0, i, 0)),
        ],
        out_specs=pl.BlockSpec((_RS, 128), lambda c, i: (c * (N // _RS) + i, 0)),
        out_shape=jax.ShapeDtypeStruct((2 * N, 128), jnp.float32),
    )(x, degp)

    agg1 = _sc_agg1(t1, src2d, dst2d)

    z2p = pl.pallas_call(
        _tc_mid_body,
        grid=(N // _RS,),
        in_specs=[
            pl.BlockSpec((NC, _RS, 128), lambda i: (0, i, 0)),
            pl.BlockSpec((NC, _RS, 8), lambda i: (0, i, 0)),
            pl.BlockSpec((D, HID), lambda i: (0, 0)),
            pl.BlockSpec((1, HID), lambda i: (0, 0)),
            pl.BlockSpec((HID, CP), lambda i: (0, 0)),
        ],
        out_specs=pl.BlockSpec((_RS, CP), lambda i: (i, 0)),
        out_shape=jax.ShapeDtypeStruct((N, CP), jnp.float32),
    )(agg1, degp, W1, b1.reshape(1, HID), W2p)

    agg2 = _sc_agg2(z2p, zeros48, src2d, dst2d)

    out = pl.pallas_call(
        _tc_out_body,
        grid=(N // _RS,),
        in_specs=[
            pl.BlockSpec((NC, _RS, CP), lambda i: (0, i, 0)),
            pl.BlockSpec((NC, _RS, 8), lambda i: (0, i, 0)),
            pl.BlockSpec((1, C), lambda i: (0, 0)),
        ],
        out_specs=pl.BlockSpec((_RS, C), lambda i: (i, 0)),
        out_shape=jax.ShapeDtypeStruct((N, C), jnp.float32),
    )(agg2, degp, b2.reshape(1, C))

    return out


# speedup vs baseline: 12.4867x; 1.1074x over previous
import functools

import jax
import jax.numpy as jnp
from jax import lax
from jax.experimental import pallas as pl
from jax.experimental.pallas import tpu as pltpu
from jax.experimental.pallas import tpu_sc as plsc

N = 10000
D = 256
HID = 512
C = 40
CP = 48
E = 160000
EPAD = 163840
NC = 2
NS = 16
SUB = 128
NPT = N // NS
TRASH = N
ACCROWS = N + 8

_mesh = plsc.VectorSubcoreMesh(
    core_axis_name="c", subcore_axis_name="s", num_cores=NC, num_subcores=NS
)

_SROWS = 2 * EPAD // SUB
_DROWS = EPAD // SUB


@functools.partial(
    pl.kernel,
    out_type=jax.ShapeDtypeStruct((NC, N, 8), jnp.float32),
    mesh=_mesh,
    compiler_params=pltpu.CompilerParams(use_tc_tiling_on_sc=False),
    scratch_types=[
        pltpu.VMEM((8, SUB), jnp.int32),
        pltpu.VMEM((SUB, 8), jnp.float32),
        pltpu.VMEM_SHARED((ACCROWS, 8), jnp.float32),
    ],
)
def _sc_deg(dst2d, zeros8, ones8, out, didx, ones_v, acc):
    c = lax.axis_index("c")
    s = lax.axis_index("s")
    w = c * NS + s
    pltpu.sync_copy(ones8, ones_v)
    pltpu.sync_copy(zeros8.at[pl.ds(s * NPT, NPT)], acc.at[pl.ds(s * NPT, NPT)])
    plsc.subcore_barrier()

    def body(j, carry):
        pltpu.sync_copy(dst2d.at[pl.ds(w * 40 + j * 8, 8)], didx)
        for k in range(8):
            pltpu.sync_copy(ones_v, acc.at[didx.at[k]], add=True)
        return carry

    lax.fori_loop(0, EPAD // (NC * NS) // (8 * SUB), body, 0)
    plsc.subcore_barrier()
    pltpu.sync_copy(acc.at[pl.ds(s * NPT, NPT)], out.at[c, pl.ds(s * NPT, NPT)])


def _edge_agg(table, src2d, dst2d, acc, sidx, didx, rows, semg,
              ngroups, gsz, src_row0, dst_row0):
    def group(g, carry):
        pltpu.sync_copy(src2d.at[pl.ds(src_row0 + g * gsz, gsz)], sidx)
        pltpu.sync_copy(dst2d.at[pl.ds(dst_row0 + g * gsz, gsz)], didx)
        descs = [None] * gsz
        descs[0] = pltpu.async_copy(table.at[sidx.at[0]], rows[0], semg[0])
        for k in range(gsz):
            descs[k].wait()
            if k + 1 < gsz:
                descs[k + 1] = pltpu.async_copy(
                    table.at[sidx.at[k + 1]], rows[(k + 1) % 2], semg[(k + 1) % 2])
            pltpu.sync_copy(rows[k % 2], acc.at[didx.at[k]], add=True)
        return carry

    lax.fori_loop(0, ngroups, group, 0)


_NT1 = EPAD // NS // SUB
_GSZ1 = 16
_NT2 = EPAD // (NC * NS) // SUB


@functools.partial(
    pl.kernel,
    out_type=jax.ShapeDtypeStruct((NC, N, 128), jnp.float32),
    mesh=_mesh,
    compiler_params=pltpu.CompilerParams(use_tc_tiling_on_sc=False),
    scratch_types=[
        pltpu.VMEM((_GSZ1, SUB), jnp.int32),
        pltpu.VMEM((_GSZ1, SUB), jnp.int32),
        [pltpu.VMEM((SUB, 128), jnp.float32)] * 2,
        [pltpu.SemaphoreType.DMA] * 2,
        pltpu.VMEM_SHARED((ACCROWS, 128), jnp.float32),
    ],
)
def _sc_agg1(t1, src2d, dst2d, out, sidx, didx, rows, semg, acc):
    c = lax.axis_index("c")
    s = lax.axis_index("s")
    pltpu.sync_copy(t1.at[pl.ds(c * N + s * NPT, NPT)], acc.at[pl.ds(s * NPT, NPT)])
    plsc.subcore_barrier()
    _edge_agg(t1, src2d, dst2d, acc, sidx, didx, rows, semg,
              _NT1 // _GSZ1, _GSZ1, c * _DROWS + s * _NT1, s * _NT1)
    plsc.subcore_barrier()
    pltpu.sync_copy(acc.at[pl.ds(s * NPT, NPT)], out.at[c, pl.ds(s * NPT, NPT)])


@functools.partial(
    pl.kernel,
    out_type=jax.ShapeDtypeStruct((NC, N, CP), jnp.float32),
    mesh=_mesh,
    compiler_params=pltpu.CompilerParams(use_tc_tiling_on_sc=False),
    scratch_types=[
        pltpu.VMEM((8, SUB), jnp.int32),
        pltpu.VMEM((8, SUB), jnp.int32),
        [pltpu.VMEM((SUB, CP), jnp.float32)] * 2,
        [pltpu.SemaphoreType.DMA] * 2,
        pltpu.VMEM_SHARED((ACCROWS, CP), jnp.float32),
    ],
)
def _sc_agg2(z2p, zeros48, src2d, dst2d, out, sidx, didx, rows, semg, acc):
    c = lax.axis_index("c")
    s = lax.axis_index("s")

    @pl.when(c == 0)
    def _():
        pltpu.sync_copy(z2p.at[pl.ds(s * NPT, NPT)], acc.at[pl.ds(s * NPT, NPT)])

    @pl.when(c != 0)
    def _():
        pltpu.sync_copy(zeros48.at[pl.ds(s * NPT, NPT)], acc.at[pl.ds(s * NPT, NPT)])

    plsc.subcore_barrier()
    _edge_agg(z2p, src2d, dst2d, acc, sidx, didx, rows, semg,
              _NT2 // 8, 8, (c * NS + s) * _NT2, (c * NS + s) * _NT2)
    plsc.subcore_barrier()
    pltpu.sync_copy(acc.at[pl.ds(s * NPT, NPT)], out.at[c, pl.ds(s * NPT, NPT)])


def _dinv_from(degp_ref):
    deg = 1.0 + degp_ref[0, :, 0:1] + degp_ref[1, :, 0:1]
    return lax.rsqrt(deg)


def _tc_scale_body(x_ref, degp_ref, out_ref):
    out_ref[...] = x_ref[...] * _dinv_from(degp_ref)


def _tc_mid_body(agg_ref, degp_ref, w1_ref, b1_ref, w2_ref, out_ref):
    dinv = _dinv_from(degp_ref)
    ax = jnp.concatenate([agg_ref[0], agg_ref[1]], axis=1) * dinv
    h = jnp.dot(ax, w1_ref[...], preferred_element_type=jnp.float32) + b1_ref[...]
    h = jnp.maximum(h, 0.0)
    out_ref[...] = jnp.dot(h, w2_ref[...], preferred_element_type=jnp.float32) * dinv


def _tc_out_body(agg_ref, degp_ref, b2_ref, out_ref):
    dinv = _dinv_from(degp_ref)
    ssum = (agg_ref[0] + agg_ref[1]) * dinv
    logits = ssum[:, :C] + b2_ref[...]
    m = jnp.max(logits, axis=1, keepdims=True)
    lse = jnp.log(jnp.sum(jnp.exp(logits - m), axis=1, keepdims=True))
    out_ref[...] = logits - m - lse


_RS = 2000


def kernel(x, edge_index, W1, b1, W2, b2):
    src = edge_index[0].astype(jnp.int32)
    dst = edge_index[1].astype(jnp.int32)
    npad = EPAD - E
    src_p = jnp.concatenate([src, jnp.zeros((npad,), jnp.int32)])
    dst_p = jnp.concatenate([dst, jnp.full((npad,), TRASH, jnp.int32)])
    src2d = jnp.concatenate([src_p, src_p + N]).reshape(_SROWS, SUB)
    dst2d = dst_p.reshape(_DROWS, SUB)
    zeros8 = jnp.zeros((N, 8), jnp.float32)
    ones8 = jnp.ones((SUB, 8), jnp.float32)
    zeros48 = jnp.zeros((N, CP), jnp.float32)
    W2p = jnp.concatenate([W2, jnp.zeros((HID, CP - C), jnp.float32)], axis=1)

    degp = _sc_deg(dst2d, zeros8, ones8)

    t1 = pl.pallas_call(
        _tc_scale_body,
        grid=(2, N // _RS),
        in_specs=[
            pl.BlockSpec((_RS, 128), lambda c, i: (i, c)),
            pl.BlockSpec((NC, _RS, 8), lambda c, i: (0, i, 0)),
        ],
        out_specs=pl.BlockSpec((_RS, 128), lambda c, i: (c * (N // _RS) + i, 0)),
        out_shape=jax.ShapeDtypeStruct((2 * N, 128), jnp.float32),
    )(x, degp)

    agg1 = _sc_agg1(t1, src2d, dst2d)

    z2p = pl.pallas_call(
        _tc_mid_body,
        grid=(N // _RS,),
        in_specs=[
            pl.BlockSpec((NC, _RS, 128), lambda i: (0, i, 0)),
            pl.BlockSpec((NC, _RS, 8), lambda i: (0, i, 0)),
            pl.BlockSpec((D, HID), lambda i: (0, 0)),
            pl.BlockSpec((1, HID), lambda i: (0, 0)),
            pl.BlockSpec((HID, CP), lambda i: (0, 0)),
        ],
        out_specs=pl.BlockSpec((_RS, CP), lambda i: (i, 0)),
        out_shape=jax.ShapeDtypeStruct((N, CP), jnp.float32),
    )(agg1, degp, W1, b1.reshape(1, HID), W2p)

    agg2 = _sc_agg2(z2p, zeros48, src2d, dst2d)

    out = pl.pallas_call(
        _tc_out_body,
        grid=(N // _RS,),
        in_specs=[
            pl.BlockSpec((NC, _RS, CP), lambda i: (0, i, 0)),
            pl.BlockSpec((NC, _RS, 8), lambda i: (0, i, 0)),
            pl.BlockSpec((1, C), lambda i: (0, 0)),
        ],
        out_specs=pl.BlockSpec((_RS, C), lambda i: (i, 0)),
        out_shape=jax.ShapeDtypeStruct((N, C), jnp.float32),
    )(agg2, degp, b2.reshape(1, C))

    return out
